# baseline (speedup 1.0000x reference)
import functools

import jax
import jax.numpy as jnp
from jax import lax
from jax.experimental import pallas as pl
from jax.experimental.pallas import tpu as pltpu
from jax.experimental.pallas import tpu_sc as plsc

N = 1_000_000
D = 32
B = 16384
PACK = 128 // D
NR = N // PACK
CH = 256


@functools.lru_cache(maxsize=None)
def _build_gather():
    info = plsc.get_sparse_core_info()
    nc, ns, L = info.num_cores, info.num_subcores, info.num_lanes
    nw = nc * ns
    b_per_w = B // nw
    n_chunks = b_per_w // CH
    mesh = plsc.VectorSubcoreMesh(core_axis_name="c", subcore_axis_name="s")

    @functools.partial(
        pl.kernel,
        mesh=mesh,
        out_type=jax.ShapeDtypeStruct((B, D), jnp.float32),
        scratch_types=[
            pltpu.VMEM((b_per_w,), jnp.int32),
            pltpu.VMEM((b_per_w,), jnp.int32),
            pltpu.VMEM((CH, 128), jnp.float32),
            pltpu.VMEM((CH, D), jnp.float32),
            pltpu.SemaphoreType.DMA,
        ],
    )
    def gather(table_hbm, idx_hbm, out_hbm, idx_v, q_v, buf_v, out_v, sem):
        wid = lax.axis_index("s") * nc + lax.axis_index("c")
        base = wid * b_per_w
        pltpu.sync_copy(idx_hbm.at[pl.ds(base, b_per_w)], idx_v)
        for j in range(b_per_w // L):
            q_v[pl.ds(j * L, L)] = idx_v[pl.ds(j * L, L)] >> 2

        for t in range(n_chunks):
            pltpu.async_copy(
                table_hbm.at[q_v.at[pl.ds(t * CH, CH)]], buf_v, sem
            ).wait()

            def body(g, carry, t=t):
                bb = g * L
                rv = (idx_v[pl.ds(t * CH + bb, L)] & 3) * D
                for k in range(L):
                    b = bb + k
                    c = rv[k]
                    out_v[b, pl.ds(0, L)] = buf_v[b, pl.ds(c, L)]
                    out_v[b, pl.ds(L, L)] = buf_v[b, pl.ds(c + L, L)]
                return carry

            lax.fori_loop(0, CH // L, body, 0)
            pltpu.sync_copy(out_v, out_hbm.at[pl.ds(base + t * CH, CH)])

    return gather


def kernel(data, i):
    table = data.reshape(NR, 128)
    return _build_gather()(table, i.astype(jnp.int32))


# speedup vs baseline: 3.1182x; 3.1182x over previous
import functools

import jax
import jax.numpy as jnp
from jax import lax
from jax.experimental import pallas as pl
from jax.experimental.pallas import tpu as pltpu
from jax.experimental.pallas import tpu_sc as plsc

N = 1_000_000
D = 32
B = 16384
G = 8


@functools.lru_cache(maxsize=None)
def _build_gather():
    info = plsc.get_sparse_core_info()
    nc, ns, L = info.num_cores, info.num_subcores, info.num_lanes
    nw = nc * ns
    b_per_w = B // nw
    mesh = plsc.VectorSubcoreMesh(core_axis_name="c", subcore_axis_name="s")

    @functools.partial(
        pl.kernel,
        mesh=mesh,
        out_type=jax.ShapeDtypeStruct((D, B), jnp.float32),
        scratch_types=[
            pltpu.VMEM((b_per_w,), jnp.int32),
            pltpu.VMEM((G, D, 128), jnp.float32),
            pltpu.VMEM((D, b_per_w), jnp.float32),
            pltpu.SemaphoreType.DMA,
        ],
        compiler_params=pltpu.CompilerParams(needs_layout_passes=False),
    )
    def gather(table_hbm, idx_hbm, out_hbm, idx_v, buf_v, out_v, sem):
        wid = lax.axis_index("s") * nc + lax.axis_index("c")
        base = wid * b_per_w
        pltpu.sync_copy(idx_hbm.at[pl.ds(base, b_per_w)], idx_v)
        rows_lo = lax.iota(jnp.int32, L)
        rows_hi = rows_lo + L

        def body(g, carry):
            gb = g * L
            iv = idx_v[pl.ds(gb, L)]
            for half in range(L // G):
                copies = []
                for k in range(G):
                    lane = half * G + k
                    start = pl.multiple_of((iv[lane] >> 7) * 128, 128)
                    copies.append(
                        pltpu.async_copy(
                            table_hbm.at[:, pl.ds(start, 128)],
                            buf_v.at[k],
                            sem,
                        )
                    )
                for k in range(G):
                    copies[k].wait()
                for k in range(G):
                    lane = half * G + k
                    w = iv[lane] & 127
                    cols = jnp.broadcast_to(w, (L,))
                    slot = jnp.broadcast_to(gb + lane, (L,))
                    lo = plsc.load_gather(buf_v.at[k], [rows_lo, cols])
                    hi = plsc.load_gather(buf_v.at[k], [rows_hi, cols])
                    plsc.store_scatter(out_v, [rows_lo, slot], lo)
                    plsc.store_scatter(out_v, [rows_hi, slot], hi)
            return carry

        lax.fori_loop(0, b_per_w // L, body, 0)
        pltpu.sync_copy(out_v, out_hbm.at[:, pl.ds(base, b_per_w)])

    return gather


def kernel(data, i):
    out_t = _build_gather()(data.T, i.astype(jnp.int32))
    return out_t.T


# speedup vs baseline: 3.8649x; 1.2395x over previous
import functools

import jax
import jax.numpy as jnp
from jax import lax
from jax.experimental import pallas as pl
from jax.experimental.pallas import tpu as pltpu
from jax.experimental.pallas import tpu_sc as plsc

N = 1_000_000
D = 32
B = 16384
G = 8


@functools.lru_cache(maxsize=None)
def _build_gather():
    info = plsc.get_sparse_core_info()
    nc, ns, L = info.num_cores, info.num_subcores, info.num_lanes
    nw = nc * ns
    b_per_w = B // nw
    n_blocks = b_per_w // L
    mesh = plsc.VectorSubcoreMesh(core_axis_name="c", subcore_axis_name="s")

    @functools.partial(
        pl.kernel,
        mesh=mesh,
        out_type=jax.ShapeDtypeStruct((D, B), jnp.float32),
        scratch_types=[
            pltpu.VMEM((b_per_w,), jnp.int32),
            pltpu.VMEM((2, G, D, 128), jnp.float32),
            pltpu.VMEM((D, 128), jnp.float32),
            pltpu.SemaphoreType.DMA,
            pltpu.SemaphoreType.DMA,
        ],
        compiler_params=pltpu.CompilerParams(needs_layout_passes=False),
    )
    def gather(table_hbm, idx_hbm, out_hbm, idx_v, buf_v, win_v, sem0, sem1):
        wid = lax.axis_index("s") * nc + lax.axis_index("c")
        base = wid * b_per_w
        pltpu.sync_copy(idx_hbm.at[pl.ds(base, b_per_w)], idx_v)
        rows_lo = lax.iota(jnp.int32, L)
        rows_hi = rows_lo + L
        sems = (sem0, sem1)

        def fire(iv, half, bank, sem):
            for k in range(G):
                lane = half * G + k
                start = pl.multiple_of((iv[lane] >> 7) * 128, 128)
                pltpu.async_copy(
                    table_hbm.at[:, pl.ds(start, 128)],
                    buf_v.at[bank, k],
                    sem,
                )

        def wait_bank(bank, sem):
            for k in range(G):
                pltpu.make_async_copy(
                    table_hbm.at[:, pl.ds(0, 128)], buf_v.at[bank, k], sem
                ).wait()

        def extract(iv, half, bank, g):
            for k in range(G):
                lane = half * G + k
                w = iv[lane] & 127
                cols = jnp.broadcast_to(w, (L,))
                col = jnp.broadcast_to((g % 8) * L + lane, (L,))
                lo = plsc.load_gather(buf_v.at[bank, k], [rows_lo, cols])
                hi = plsc.load_gather(buf_v.at[bank, k], [rows_hi, cols])
                plsc.store_scatter(win_v, [rows_lo, col], lo)
                plsc.store_scatter(win_v, [rows_hi, col], hi)
            if half == 1:
                @pl.when(g % 8 == 7)
                def _():
                    t = g // 8
                    pltpu.sync_copy(
                        win_v, out_hbm.at[:, pl.ds(base + t * 128, 128)]
                    )

        iv0 = idx_v[pl.ds(0, L)]
        fire(iv0, 0, 1, sem1)
        fire(iv0, 1, 0, sem0)
        wait_bank(1, sem1)
        extract(iv0, 0, 1, 0)

        def body(g, iv_prev):
            iv = idx_v[pl.ds(g * L, L)]
            fire(iv, 0, 1, sem1)
            wait_bank(0, sem0)
            extract(iv_prev, 1, 0, g - 1)
            fire(iv, 1, 0, sem0)
            wait_bank(1, sem1)
            extract(iv, 0, 1, g)
            return iv

        iv_last = lax.fori_loop(1, n_blocks, body, iv0)
        wait_bank(0, sem0)
        extract(iv_last, 1, 0, n_blocks - 1)

    return gather


def kernel(data, i):
    out_t = _build_gather()(data.T, i.astype(jnp.int32))
    return out_t.T
